# baseline (speedup 1.0000x reference)
import functools

import jax
import jax.numpy as jnp
from jax import lax
from jax.experimental import pallas as pl
from jax.experimental.pallas import tpu as pltpu
from jax.experimental.pallas import tpu_sc as plsc

_LANES = 16
_CHUNK = 128


@functools.cache
def _build(B, D, n_users, n_items):
    info = plsc.get_sparse_core_info()
    NC, NS = info.num_cores, info.num_subcores
    NW = NC * NS
    assert B % (NW * _CHUNK) == 0 and D % _LANES == 0
    b_per_w = B // NW
    n_chunks = b_per_w // _CHUNK

    mesh = plsc.VectorSubcoreMesh(core_axis_name="c", subcore_axis_name="s")

    @functools.partial(
        pl.kernel,
        mesh=mesh,
        compiler_params=pltpu.CompilerParams(use_tc_tiling_on_sc=False),
        out_type=jax.ShapeDtypeStruct((B, D), jnp.float32),
        scratch_types=[
            pltpu.VMEM((n_chunks, _CHUNK), jnp.int32),
            pltpu.VMEM((n_chunks, _CHUNK), jnp.int32),
            pltpu.VMEM((b_per_w, D), jnp.float32),
            pltpu.VMEM((b_per_w, D), jnp.float32),
            pltpu.SemaphoreType.DMA,
        ],
    )
    def gmf(uid_hbm, iid_hbm, ut_hbm, it_hbm, out_hbm,
            uidx, iidx, urows, irows, sem):
        wid = lax.axis_index("s") * NC + lax.axis_index("c")
        base = wid * b_per_w
        for c in range(n_chunks):
            pltpu.sync_copy(uid_hbm.at[pl.ds(base + c * _CHUNK, _CHUNK)],
                            uidx.at[c])
            pltpu.sync_copy(iid_hbm.at[pl.ds(base + c * _CHUNK, _CHUNK)],
                            iidx.at[c])
        copies = []
        for c in range(n_chunks):
            copies.append(pltpu.async_copy(
                ut_hbm.at[uidx.at[c]], urows.at[pl.ds(c * _CHUNK, _CHUNK)],
                sem))
            copies.append(pltpu.async_copy(
                it_hbm.at[iidx.at[c]], irows.at[pl.ds(c * _CHUNK, _CHUNK)],
                sem))
        for cp in copies:
            cp.wait()

        def mul_row(r, carry):
            for h in range(0, D, _LANES):
                u = urows[r, pl.ds(h, _LANES)]
                v = irows[r, pl.ds(h, _LANES)]
                urows[r, pl.ds(h, _LANES)] = u * v
            return carry

        lax.fori_loop(0, b_per_w, mul_row, 0, unroll=4)
        pltpu.sync_copy(urows, out_hbm.at[pl.ds(base, b_per_w)])

    return gmf


def kernel(user_ids, item_ids, user_table, item_table):
    B, = user_ids.shape
    n_users, D = user_table.shape
    n_items = item_table.shape[0]
    gmf = _build(B, D, n_users, n_items)
    return gmf(user_ids.astype(jnp.int32), item_ids.astype(jnp.int32),
               user_table, item_table)
